# baseline (speedup 1.0000x reference)
import functools
import jax
import jax.numpy as jnp
import numpy as np
from jax import lax
from jax.experimental import pallas as pl
from jax.experimental.pallas import tpu as pltpu
from jax.experimental.pallas import tpu_sc as plsc

N = 10000
E = 160000
C = 32
H = 8
S = 32
M = 9
NPAD = 10240
NB_N = NPAD // 512
EB = 1280
NB_E = E // EB
_DEG_IDX = np.array([0, 1, 1, 1, 2, 2, 2, 2, 2])

_NC = 2
_NS = 16
_CH = 128
_NCHUNK = E // _CH
_CH4 = 80
_NCHUNK4 = E // _CH4
_HC = M * C // 2
_STRIPE = NPAD // _NS

_MESH = plsc.VectorSubcoreMesh(core_axis_name="c", subcore_axis_name="s")


def _ln(x, g, b):
    mu = x.mean(-1, keepdims=True)
    var = ((x - mu) ** 2).mean(-1, keepdims=True)
    return (x - mu) / jnp.sqrt(var + 1e-5) * g + b


def _k1_body(flat_ref, at_ref, wq_ref, bq_ref, wk_ref, bk_ref, ts_ref, td_ref,
             q_ref, k_ref, ps_ref, pd_ref):
    flat = flat_ref[...]
    q_ref[...] = jnp.dot(flat, wq_ref[...], preferred_element_type=jnp.float32) + bq_ref[...]
    k_ref[...] = jnp.dot(flat, wk_ref[...], preferred_element_type=jnp.float32) + bk_ref[...]
    at = at_ref[0]
    oh = (at.reshape(512, 1) == jax.lax.broadcasted_iota(jnp.int32, (512, 256), 1)).astype(jnp.float32)
    ps_ref[...] = jnp.dot(oh, ts_ref[...], preferred_element_type=jnp.float32)
    pd_ref[...] = jnp.dot(oh, td_ref[...], preferred_element_type=jnp.float32)


def _k2_body(q_hbm, k_hbm, src_hbm, dst_hbm,
             qd_out, ks_out,
             idxs, idxd, qrows, krows, sem):
    cid = lax.axis_index("c")
    sid = lax.axis_index("s")
    w = sid * _NC + cid
    nw = _NC * _NS
    niter = (_NCHUNK - 1 - w) // nw + 1

    def body(i, carry):
        base = (w + i * nw) * _CH
        pltpu.sync_copy(src_hbm.at[pl.ds(base, _CH)], idxs)
        pltpu.sync_copy(dst_hbm.at[pl.ds(base, _CH)], idxd)
        cp1 = pltpu.async_copy(q_hbm.at[idxd], qrows, sem)
        cp2 = pltpu.async_copy(k_hbm.at[idxs], krows, sem)
        cp1.wait()
        cp2.wait()
        pltpu.sync_copy(qrows, qd_out.at[pl.ds(base, _CH)])
        pltpu.sync_copy(krows, ks_out.at[pl.ds(base, _CH)])
        return carry

    lax.fori_loop(0, niter, body, 0)


_k2_call = functools.partial(
    pl.kernel,
    out_type=[
        jax.ShapeDtypeStruct((E, H * S), jnp.float32),
        jax.ShapeDtypeStruct((E, H * S), jnp.float32),
    ],
    mesh=_MESH,
    scratch_types=[
        pltpu.VMEM((_CH,), jnp.int32),
        pltpu.VMEM((_CH,), jnp.int32),
        pltpu.VMEM((_CH, H * S), jnp.float32),
        pltpu.VMEM((_CH, H * S), jnp.float32),
        pltpu.SemaphoreType.DMA,
    ],
)(_k2_body)


def _k2b_body(ps_hbm, pd_hbm, src_hbm, dst_hbm,
              pse_out, pde_out,
              idxs, idxd, psrows, pdrows, sem):
    cid = lax.axis_index("c")
    sid = lax.axis_index("s")
    w = sid * _NC + cid
    nw = _NC * _NS
    niter = (_NCHUNK - 1 - w) // nw + 1

    def body(i, carry):
        base = (w + i * nw) * _CH
        pltpu.sync_copy(src_hbm.at[pl.ds(base, _CH)], idxs)
        pltpu.sync_copy(dst_hbm.at[pl.ds(base, _CH)], idxd)
        cp1 = pltpu.async_copy(ps_hbm.at[idxs], psrows, sem)
        cp2 = pltpu.async_copy(pd_hbm.at[idxd], pdrows, sem)
        cp1.wait()
        cp2.wait()
        pltpu.sync_copy(psrows, pse_out.at[pl.ds(base, _CH)])
        pltpu.sync_copy(pdrows, pde_out.at[pl.ds(base, _CH)])
        return carry

    lax.fori_loop(0, niter, body, 0)


_k2b_call = functools.partial(
    pl.kernel,
    out_type=[
        jax.ShapeDtypeStruct((E, 32), jnp.float32),
        jax.ShapeDtypeStruct((E, 32), jnp.float32),
    ],
    mesh=_MESH,
    scratch_types=[
        pltpu.VMEM((_CH,), jnp.int32),
        pltpu.VMEM((_CH,), jnp.int32),
        pltpu.VMEM((_CH, 32), jnp.float32),
        pltpu.VMEM((_CH, 32), jnp.float32),
        pltpu.SemaphoreType.DMA,
    ],
    compiler_params=pltpu.CompilerParams(use_tc_tiling_on_sc=False),
)(_k2b_body)


def _k3_body(aw4_ref, pse4_ref, pde4_ref, qd4_ref, ks4_ref, pedge4_ref,
             w1a4_ref, b1c4_ref, mones_ref, g14_ref, be14_ref, g24_ref, be24_ref,
             w2bd4_ref, w3eb4_ref, b3eb4_ref, w3wv4_ref, b3wv4_ref,
             adm4_ref, pexp_ref, rrep4_ref,
             ex_ref, coeff_ref):
    mones = mones_ref[...]

    def ln4(h, g, b):
        mu = jnp.dot(h, mones, preferred_element_type=jnp.float32)
        d = h - mu
        var = jnp.dot(d * d, mones, preferred_element_type=jnp.float32)
        return d * jax.lax.rsqrt(var + 1e-5) * g + b

    h1 = jnp.dot(aw4_ref[...], w1a4_ref[...], preferred_element_type=jnp.float32) \
        + pse4_ref[...] + pde4_ref[...] + b1c4_ref[...]
    s1 = jax.nn.silu(ln4(h1, g14_ref[...], be14_ref[...]))
    h2 = jnp.dot(s1, w2bd4_ref[...], preferred_element_type=jnp.float32)
    s2 = jax.nn.silu(ln4(h2, g24_ref[...], be24_ref[...]))
    eb4 = jnp.dot(s2, w3eb4_ref[...], preferred_element_type=jnp.float32) + b3eb4_ref[...]
    wv4 = jnp.dot(s2, w3wv4_ref[...], preferred_element_type=jnp.float32) + b3wv4_ref[...]
    qk = qd4_ref[...] * ks4_ref[...]
    pre = 0.6 * qk + 0.4 * qk * (2.0 * jax.nn.sigmoid(qk) - 1.0)
    alpha4 = jnp.dot(pre, adm4_ref[...], preferred_element_type=jnp.float32) \
        + eb4 + jnp.dot(pedge4_ref[...], pexp_ref[...], preferred_element_type=jnp.float32)
    ex4 = jnp.exp(alpha4)
    ex_ref[...] = ex4
    coeff_ref[...] = wv4 * jnp.dot(ex4, rrep4_ref[...], preferred_element_type=jnp.float32)


def _k4_body(nf_lo, nf_hi, coeff_hbm, ex_hbm, src_hbm, dst_hbm, zagg_hbm, zden_hbm,
             agg_lo_out, agg_hi_out, den_out,
             aggsh, densh,
             idxs0, idxd0, nfrows0, coeffv0,
             idxs1, idxd1, nfrows1, coeffv1,
             exv, gsem0, gsem1):
    cid = lax.axis_index("c")
    sid = lax.axis_index("s")
    pltpu.sync_copy(zagg_hbm, aggsh.at[pl.ds(sid * _STRIPE, _STRIPE)])

    @pl.when(cid == 0)
    def _():
        pltpu.sync_copy(zden_hbm, densh.at[pl.ds(sid * _STRIPE, _STRIPE)])

    plsc.subcore_barrier()

    def prefetch(c, idxs, idxd, nfrows, coeffv, gsem):
        base = c * _CH4
        pltpu.sync_copy(src_hbm.at[pl.ds(base, _CH4)], idxs)
        pltpu.sync_copy(dst_hbm.at[pl.ds(base, _CH4)], idxd)

        @pl.when(cid == 0)
        def _():
            pltpu.async_copy(nf_lo.at[idxs], nfrows, gsem)

        @pl.when(cid == 1)
        def _():
            pltpu.async_copy(nf_hi.at[idxs], nfrows, gsem)

        pltpu.sync_copy(coeff_hbm.at[pl.ds(base, _CH4)], coeffv)

    def process(c, idxs, idxd, nfrows, coeffv, gsem):
        base = c * _CH4

        @pl.when(cid == 0)
        def _():
            pltpu.make_async_copy(nf_lo.at[idxs], nfrows, gsem).wait()

        @pl.when(cid == 1)
        def _():
            pltpu.make_async_copy(nf_hi.at[idxs], nfrows, gsem).wait()

        @plsc.parallel_loop(0, _CH4, step=1, unroll=4)
        def _(e):
            ca = coeffv[e, pl.ds(0, 16)]
            cb = coeffv[e, pl.ds(16, 16)]
            ma = jnp.where(cid == 0, ca, cb)
            mb = jnp.where(cid == 0, cb, ca)
            for j in range(M):
                mult = ma if j % 2 == 0 else mb
                nfrows[e, pl.ds(j * 16, 16)] = nfrows[e, pl.ds(j * 16, 16)] * mult

        pltpu.sync_copy(nfrows, aggsh.at[idxd], add=True)

        @pl.when(cid == 0)
        def _():
            pltpu.sync_copy(ex_hbm.at[pl.ds(base, _CH4)], exv)
            pltpu.sync_copy(exv, densh.at[idxd], add=True)

    nouter = (_NCHUNK4 // _NS + 2) // 2

    def body(i, carry):
        c0 = sid + (2 * i) * _NS
        c1 = sid + (2 * i + 1) * _NS
        v0 = c0 < _NCHUNK4
        v1 = c1 < _NCHUNK4

        @pl.when(v0)
        def _():
            prefetch(c0, idxs0, idxd0, nfrows0, coeffv0, gsem0)

        @pl.when(v1)
        def _():
            prefetch(c1, idxs1, idxd1, nfrows1, coeffv1, gsem1)

        @pl.when(v0)
        def _():
            process(c0, idxs0, idxd0, nfrows0, coeffv0, gsem0)

        @pl.when(v1)
        def _():
            process(c1, idxs1, idxd1, nfrows1, coeffv1, gsem1)

        return carry

    lax.fori_loop(0, nouter, body, 0)
    plsc.subcore_barrier()
    row0 = sid * _STRIPE

    @pl.when(cid == 0)
    def _():
        pltpu.sync_copy(aggsh.at[pl.ds(row0, _STRIPE)], agg_lo_out.at[pl.ds(row0, _STRIPE)])
        pltpu.sync_copy(densh.at[pl.ds(row0, _STRIPE)], den_out.at[pl.ds(row0, _STRIPE)])

    @pl.when(cid == 1)
    def _():
        pltpu.sync_copy(aggsh.at[pl.ds(row0, _STRIPE)], agg_hi_out.at[pl.ds(row0, _STRIPE)])


_k4_call = functools.partial(
    pl.kernel,
    out_type=[
        jax.ShapeDtypeStruct((NPAD, _HC), jnp.float32),
        jax.ShapeDtypeStruct((NPAD, _HC), jnp.float32),
        jax.ShapeDtypeStruct((NPAD, 8), jnp.float32),
    ],
    mesh=_MESH,
    scratch_types=[
        pltpu.VMEM_SHARED((NPAD, _HC), jnp.float32),
        pltpu.VMEM_SHARED((NPAD, 8), jnp.float32),
        pltpu.VMEM((_CH4,), jnp.int32),
        pltpu.VMEM((_CH4,), jnp.int32),
        pltpu.VMEM((_CH4, _HC), jnp.float32),
        pltpu.VMEM((_CH4, 32), jnp.float32),
        pltpu.VMEM((_CH4,), jnp.int32),
        pltpu.VMEM((_CH4,), jnp.int32),
        pltpu.VMEM((_CH4, _HC), jnp.float32),
        pltpu.VMEM((_CH4, 32), jnp.float32),
        pltpu.VMEM((_CH4, 8), jnp.float32),
        pltpu.SemaphoreType.DMA,
        pltpu.SemaphoreType.DMA,
    ],
    compiler_params=pltpu.CompilerParams(use_tc_tiling_on_sc=False),
)(_k4_body)


def _k5_body(agglo_ref, agghi_ref, den_ref, wbd_ref, bflat_ref, out_ref):
    den = den_ref[...]
    dexp = jnp.tile(jnp.repeat(den, C // H, axis=1), (1, M))
    aggu = jnp.concatenate([agglo_ref[...], agghi_ref[...]], axis=1)
    agg = aggu / (dexp + 1e-9)
    out_ref[...] = jnp.dot(agg, wbd_ref[...], preferred_element_type=jnp.float32) + bflat_ref[...]


def kernel(node_feat, attn_weight, edge_dis, edge_index, atom_types, params):
    p = params
    src = edge_index[0].astype(jnp.int32)
    dst = edge_index[1].astype(jnp.int32)

    w1f, w1r = p['fce_w1'], p['rad_w1']
    ts = p['src_emb'] @ jnp.concatenate([w1f[32:160], w1r[32:160]], axis=1)
    td = p['dst_emb'] @ jnp.concatenate([w1f[160:288], w1r[160:288]], axis=1)
    w1a = jnp.concatenate([w1f[:32], w1r[:32]], axis=1)
    eye4 = jnp.eye(4, dtype=jnp.float32)
    w1a4 = jnp.kron(eye4, w1a)
    b1c4 = jnp.tile(jnp.concatenate([p['fce_b1'], p['rad_b1']]), 4).reshape(1, 128)
    mones = jnp.kron(jnp.eye(8, dtype=jnp.float32),
                     jnp.full((16, 16), 1.0 / 16.0, jnp.float32))
    g14 = jnp.tile(jnp.concatenate([p['fce_g1'], p['rad_g1']]), 4).reshape(1, 128)
    be14 = jnp.tile(jnp.concatenate([p['fce_be1'], p['rad_be1']]), 4).reshape(1, 128)
    g24 = jnp.tile(jnp.concatenate([p['fce_g2'], p['rad_g2']]), 4).reshape(1, 128)
    be24 = jnp.tile(jnp.concatenate([p['fce_be2'], p['rad_be2']]), 4).reshape(1, 128)
    w2bd = jnp.zeros((32, 32), jnp.float32).at[:16, :16].set(p['fce_w2']).at[16:, 16:].set(p['rad_w2'])
    w2bd4 = jnp.kron(eye4, w2bd)
    w3eb = jnp.zeros((32, 8), jnp.float32).at[:16, :].set(p['fce_w3'])
    w3eb4 = jnp.kron(eye4, w3eb)
    b3eb4 = jnp.tile(p['fce_b3'], 4).reshape(1, 32)
    w3wv = jnp.zeros((32, 32), jnp.float32).at[16:, :].set(p['rad_w3'])
    w3wv4 = jnp.kron(eye4, w3wv)
    b3wv4 = jnp.tile(p['rad_b3'], 4).reshape(1, 128)
    rrep = jnp.kron(jnp.eye(H, dtype=jnp.float32),
                    jnp.ones((1, C // H), jnp.float32))
    rrep4 = jnp.kron(eye4, rrep)
    pexp = jnp.kron(eye4, jnp.ones((1, 8), jnp.float32))
    ad = p['alpha_dot'] / np.sqrt(S)
    adm = (ad[:, None, :] * jnp.eye(H, dtype=jnp.float32)[:, :, None]
           ).transpose(0, 2, 1).reshape(H * S, H)
    adm4 = jnp.kron(eye4, adm)
    wpm = p['wproj'][_DEG_IDX]
    bpm = p['bproj'][_DEG_IDX]
    eye_m = jnp.eye(M, dtype=jnp.float32)
    wbd = (eye_m[:, None, :, None] * wpm[:, :, None, :]).reshape(M * C, M * C)
    bflat = bpm.reshape(1, M * C)

    flat = node_feat.reshape(N, M * C)
    flat_p = jnp.pad(flat, ((0, NPAD - N), (0, 0)))
    at_p = jnp.pad(atom_types.astype(jnp.int32), (0, NPAD - N)).reshape(NB_N, 1, 512)

    q, k, ps, pd = pl.pallas_call(
        _k1_body,
        grid=(NB_N,),
        in_specs=[
            pl.BlockSpec((512, M * C), lambda i: (i, 0)),
            pl.BlockSpec((1, 1, 512), lambda i: (i, 0, 0)),
            pl.BlockSpec((M * C, H * S), lambda i: (0, 0)),
            pl.BlockSpec((1, H * S), lambda i: (0, 0)),
            pl.BlockSpec((M * C, H * S), lambda i: (0, 0)),
            pl.BlockSpec((1, H * S), lambda i: (0, 0)),
            pl.BlockSpec((256, 32), lambda i: (0, 0)),
            pl.BlockSpec((256, 32), lambda i: (0, 0)),
        ],
        out_specs=[
            pl.BlockSpec((512, H * S), lambda i: (i, 0)),
            pl.BlockSpec((512, H * S), lambda i: (i, 0)),
            pl.BlockSpec((512, 32), lambda i: (i, 0)),
            pl.BlockSpec((512, 32), lambda i: (i, 0)),
        ],
        out_shape=[
            jax.ShapeDtypeStruct((NPAD, H * S), jnp.float32),
            jax.ShapeDtypeStruct((NPAD, H * S), jnp.float32),
            jax.ShapeDtypeStruct((NPAD, 32), jnp.float32),
            jax.ShapeDtypeStruct((NPAD, 32), jnp.float32),
        ],
    )(flat_p, at_p, p['wq'], p['bq'].reshape(1, -1), p['wk'], p['bk'].reshape(1, -1), ts, td)

    pse, pde = _k2b_call(ps, pd, src, dst)
    qd, ks = _k2_call(q, k, src, dst)

    pedge4 = (p['poly'][0] + p['poly'][1] * edge_dis).reshape(E // 4, 4)
    aw4 = attn_weight.reshape(E // 4, 128)
    pse4 = pse.reshape(E // 4, 128)
    pde4 = pde.reshape(E // 4, 128)
    qd4 = qd.reshape(E // 4, 4 * H * S)
    ks4 = ks.reshape(E // 4, 4 * H * S)
    ex4, coeff4 = pl.pallas_call(
        _k3_body,
        grid=(NB_E,),
        in_specs=[
            pl.BlockSpec((EB // 4, 128), lambda i: (i, 0)),
            pl.BlockSpec((EB // 4, 128), lambda i: (i, 0)),
            pl.BlockSpec((EB // 4, 128), lambda i: (i, 0)),
            pl.BlockSpec((EB // 4, 4 * H * S), lambda i: (i, 0)),
            pl.BlockSpec((EB // 4, 4 * H * S), lambda i: (i, 0)),
            pl.BlockSpec((EB // 4, 4), lambda i: (i, 0)),
            pl.BlockSpec((128, 128), lambda i: (0, 0)),
            pl.BlockSpec((1, 128), lambda i: (0, 0)),
            pl.BlockSpec((128, 128), lambda i: (0, 0)),
            pl.BlockSpec((1, 128), lambda i: (0, 0)),
            pl.BlockSpec((1, 128), lambda i: (0, 0)),
            pl.BlockSpec((1, 128), lambda i: (0, 0)),
            pl.BlockSpec((1, 128), lambda i: (0, 0)),
            pl.BlockSpec((128, 128), lambda i: (0, 0)),
            pl.BlockSpec((128, 32), lambda i: (0, 0)),
            pl.BlockSpec((1, 32), lambda i: (0, 0)),
            pl.BlockSpec((128, 128), lambda i: (0, 0)),
            pl.BlockSpec((1, 128), lambda i: (0, 0)),
            pl.BlockSpec((4 * H * S, 32), lambda i: (0, 0)),
            pl.BlockSpec((4, 32), lambda i: (0, 0)),
            pl.BlockSpec((32, 128), lambda i: (0, 0)),
        ],
        out_specs=[
            pl.BlockSpec((EB // 4, 32), lambda i: (i, 0)),
            pl.BlockSpec((EB // 4, 128), lambda i: (i, 0)),
        ],
        out_shape=[
            jax.ShapeDtypeStruct((E // 4, 32), jnp.float32),
            jax.ShapeDtypeStruct((E // 4, 128), jnp.float32),
        ],
    )(aw4, pse4, pde4, qd4, ks4, pedge4, w1a4, b1c4, mones, g14, be14, g24, be24,
      w2bd4, w3eb4, b3eb4, w3wv4, b3wv4, adm4, pexp, rrep4)
    ex = ex4.reshape(E, 8)
    coeff = coeff4.reshape(E, 32)

    nf_lo = flat[:, :_HC]
    nf_hi = flat[:, _HC:]
    zagg = jnp.zeros((_STRIPE, _HC), jnp.float32)
    zden = jnp.zeros((_STRIPE, 8), jnp.float32)
    agg_lo, agg_hi, den = _k4_call(nf_lo, nf_hi, coeff, ex, src, dst, zagg, zden)

    out = pl.pallas_call(
        _k5_body,
        grid=(NB_N,),
        in_specs=[
            pl.BlockSpec((512, _HC), lambda i: (i, 0)),
            pl.BlockSpec((512, _HC), lambda i: (i, 0)),
            pl.BlockSpec((512, 8), lambda i: (i, 0)),
            pl.BlockSpec((M * C, M * C), lambda i: (0, 0)),
            pl.BlockSpec((1, M * C), lambda i: (0, 0)),
        ],
        out_specs=pl.BlockSpec((512, M * C), lambda i: (i, 0)),
        out_shape=jax.ShapeDtypeStruct((NPAD, M * C), jnp.float32),
    )(agg_lo, agg_hi, den, wbd, bflat)

    return out[:N].reshape(N, M, C)


# speedup vs baseline: 1.0297x; 1.0297x over previous
import functools
import jax
import jax.numpy as jnp
import numpy as np
from jax import lax
from jax.experimental import pallas as pl
from jax.experimental.pallas import tpu as pltpu
from jax.experimental.pallas import tpu_sc as plsc

N = 10000
E = 160000
C = 32
H = 8
S = 32
M = 9
NPAD = 10240
NB_N = NPAD // 512
EB = 1280
NB_E = E // EB
_DEG_IDX = np.array([0, 1, 1, 1, 2, 2, 2, 2, 2])

_NC = 2
_NS = 16
_CH = 128
_NCHUNK = E // _CH
_HC = M * C // 2
_STRIPE = NPAD // _NS

_MESH = plsc.VectorSubcoreMesh(core_axis_name="c", subcore_axis_name="s")


def _ln(x, g, b):
    mu = x.mean(-1, keepdims=True)
    var = ((x - mu) ** 2).mean(-1, keepdims=True)
    return (x - mu) / jnp.sqrt(var + 1e-5) * g + b


def _k1_body(flat_ref, at_ref, wq_ref, bq_ref, wk_ref, bk_ref, ts_ref, td_ref,
             q_ref, k_ref, ps_ref, pd_ref):
    flat = flat_ref[...]
    q_ref[...] = jnp.dot(flat, wq_ref[...], preferred_element_type=jnp.float32) + bq_ref[...]
    k_ref[...] = jnp.dot(flat, wk_ref[...], preferred_element_type=jnp.float32) + bk_ref[...]
    at = at_ref[0]
    oh = (at.reshape(512, 1) == jax.lax.broadcasted_iota(jnp.int32, (512, 256), 1)).astype(jnp.float32)
    ps_ref[...] = jnp.dot(oh, ts_ref[...], preferred_element_type=jnp.float32)
    pd_ref[...] = jnp.dot(oh, td_ref[...], preferred_element_type=jnp.float32)


def _k2_body(q_hbm, k_hbm, src_hbm, dst_hbm,
             qd_out, ks_out,
             idxs, idxd, qrows, krows, sem):
    cid = lax.axis_index("c")
    sid = lax.axis_index("s")
    w = sid * _NC + cid
    nw = _NC * _NS
    niter = (_NCHUNK - 1 - w) // nw + 1

    def body(i, carry):
        base = (w + i * nw) * _CH
        pltpu.sync_copy(src_hbm.at[pl.ds(base, _CH)], idxs)
        pltpu.sync_copy(dst_hbm.at[pl.ds(base, _CH)], idxd)
        cp1 = pltpu.async_copy(q_hbm.at[idxd], qrows, sem)
        cp2 = pltpu.async_copy(k_hbm.at[idxs], krows, sem)
        cp1.wait()
        cp2.wait()
        pltpu.sync_copy(qrows, qd_out.at[pl.ds(base, _CH)])
        pltpu.sync_copy(krows, ks_out.at[pl.ds(base, _CH)])
        return carry

    lax.fori_loop(0, niter, body, 0)


_k2_call = functools.partial(
    pl.kernel,
    out_type=[
        jax.ShapeDtypeStruct((E, H * S), jnp.float32),
        jax.ShapeDtypeStruct((E, H * S), jnp.float32),
    ],
    mesh=_MESH,
    scratch_types=[
        pltpu.VMEM((_CH,), jnp.int32),
        pltpu.VMEM((_CH,), jnp.int32),
        pltpu.VMEM((_CH, H * S), jnp.float32),
        pltpu.VMEM((_CH, H * S), jnp.float32),
        pltpu.SemaphoreType.DMA,
    ],
)(_k2_body)


def _k2b_body(ps_hbm, pd_hbm, src_hbm, dst_hbm,
              pse_out, pde_out,
              idxs, idxd, psrows, pdrows, sem):
    cid = lax.axis_index("c")
    sid = lax.axis_index("s")
    w = sid * _NC + cid
    nw = _NC * _NS
    niter = (_NCHUNK - 1 - w) // nw + 1

    def body(i, carry):
        base = (w + i * nw) * _CH
        pltpu.sync_copy(src_hbm.at[pl.ds(base, _CH)], idxs)
        pltpu.sync_copy(dst_hbm.at[pl.ds(base, _CH)], idxd)
        cp1 = pltpu.async_copy(ps_hbm.at[idxs], psrows, sem)
        cp2 = pltpu.async_copy(pd_hbm.at[idxd], pdrows, sem)
        cp1.wait()
        cp2.wait()
        pltpu.sync_copy(psrows, pse_out.at[pl.ds(base, _CH)])
        pltpu.sync_copy(pdrows, pde_out.at[pl.ds(base, _CH)])
        return carry

    lax.fori_loop(0, niter, body, 0)


_k2b_call = functools.partial(
    pl.kernel,
    out_type=[
        jax.ShapeDtypeStruct((E, 32), jnp.float32),
        jax.ShapeDtypeStruct((E, 32), jnp.float32),
    ],
    mesh=_MESH,
    scratch_types=[
        pltpu.VMEM((_CH,), jnp.int32),
        pltpu.VMEM((_CH,), jnp.int32),
        pltpu.VMEM((_CH, 32), jnp.float32),
        pltpu.VMEM((_CH, 32), jnp.float32),
        pltpu.SemaphoreType.DMA,
    ],
    compiler_params=pltpu.CompilerParams(use_tc_tiling_on_sc=False),
)(_k2b_body)


def _k3_body(aw4_ref, pse4_ref, pde4_ref, qd4_ref, ks4_ref, pedge4_ref,
             w1a4_ref, b1c4_ref, mones_ref, g14_ref, be14_ref, g24_ref, be24_ref,
             w2bd4_ref, w3eb4_ref, b3eb4_ref, w3wv4_ref, b3wv4_ref,
             adm4_ref, pexp_ref, rrep4_ref,
             ex_ref, coeff_ref):
    mones = mones_ref[...]

    def ln4(h, g, b):
        mu = jnp.dot(h, mones, preferred_element_type=jnp.float32)
        d = h - mu
        var = jnp.dot(d * d, mones, preferred_element_type=jnp.float32)
        return d * jax.lax.rsqrt(var + 1e-5) * g + b

    h1 = jnp.dot(aw4_ref[...], w1a4_ref[...], preferred_element_type=jnp.float32) \
        + pse4_ref[...] + pde4_ref[...] + b1c4_ref[...]
    s1 = jax.nn.silu(ln4(h1, g14_ref[...], be14_ref[...]))
    h2 = jnp.dot(s1, w2bd4_ref[...], preferred_element_type=jnp.float32)
    s2 = jax.nn.silu(ln4(h2, g24_ref[...], be24_ref[...]))
    eb4 = jnp.dot(s2, w3eb4_ref[...], preferred_element_type=jnp.float32) + b3eb4_ref[...]
    wv4 = jnp.dot(s2, w3wv4_ref[...], preferred_element_type=jnp.float32) + b3wv4_ref[...]
    qk = qd4_ref[...] * ks4_ref[...]
    pre = 0.6 * qk + 0.4 * qk * (2.0 * jax.nn.sigmoid(qk) - 1.0)
    alpha4 = jnp.dot(pre, adm4_ref[...], preferred_element_type=jnp.float32) \
        + eb4 + jnp.dot(pedge4_ref[...], pexp_ref[...], preferred_element_type=jnp.float32)
    ex4 = jnp.exp(alpha4)
    ex_ref[...] = ex4
    coeff_ref[...] = wv4 * jnp.dot(ex4, rrep4_ref[...], preferred_element_type=jnp.float32)


def _k4_body(nf_lo, nf_hi, coeff_hbm, ex_hbm, src_hbm, dst_hbm, zagg_hbm, zden_hbm,
             agg_lo_out, agg_hi_out, den_out,
             aggsh, densh,
             idxs0, idxd0, nfrows0, coeffv0,
             exv, gsem0):
    cid = lax.axis_index("c")
    sid = lax.axis_index("s")
    pltpu.sync_copy(zagg_hbm, aggsh.at[pl.ds(sid * _STRIPE, _STRIPE)])

    @pl.when(cid == 0)
    def _():
        pltpu.sync_copy(zden_hbm, densh.at[pl.ds(sid * _STRIPE, _STRIPE)])

    plsc.subcore_barrier()

    def prefetch(c, idxs, idxd, nfrows, coeffv, gsem):
        base = c * _CH
        pltpu.sync_copy(src_hbm.at[pl.ds(base, _CH)], idxs)
        pltpu.sync_copy(dst_hbm.at[pl.ds(base, _CH)], idxd)

        @pl.when(cid == 0)
        def _():
            pltpu.async_copy(nf_lo.at[idxs], nfrows, gsem)

        @pl.when(cid == 1)
        def _():
            pltpu.async_copy(nf_hi.at[idxs], nfrows, gsem)

        pltpu.sync_copy(coeff_hbm.at[pl.ds(base, _CH)], coeffv)

    def process(c, idxs, idxd, nfrows, coeffv, gsem):
        base = c * _CH

        @pl.when(cid == 0)
        def _():
            pltpu.make_async_copy(nf_lo.at[idxs], nfrows, gsem).wait()

        @pl.when(cid == 1)
        def _():
            pltpu.make_async_copy(nf_hi.at[idxs], nfrows, gsem).wait()

        @plsc.parallel_loop(0, _CH, step=1, unroll=4)
        def _(e):
            ca = coeffv[e, pl.ds(0, 16)]
            cb = coeffv[e, pl.ds(16, 16)]
            ma = jnp.where(cid == 0, ca, cb)
            mb = jnp.where(cid == 0, cb, ca)
            for j in range(M):
                mult = ma if j % 2 == 0 else mb
                nfrows[e, pl.ds(j * 16, 16)] = nfrows[e, pl.ds(j * 16, 16)] * mult

        pltpu.sync_copy(nfrows, aggsh.at[idxd], add=True)

        @pl.when(cid == 0)
        def _():
            pltpu.sync_copy(ex_hbm.at[pl.ds(base, _CH)], exv)
            pltpu.sync_copy(exv, densh.at[idxd], add=True)

    niter = (_NCHUNK - 1 - sid) // _NS + 1

    def body(i, carry):
        c0 = sid + i * _NS
        prefetch(c0, idxs0, idxd0, nfrows0, coeffv0, gsem0)
        process(c0, idxs0, idxd0, nfrows0, coeffv0, gsem0)
        return carry

    lax.fori_loop(0, niter, body, 0)
    plsc.subcore_barrier()
    row0 = sid * _STRIPE

    @pl.when(cid == 0)
    def _():
        pltpu.sync_copy(aggsh.at[pl.ds(row0, _STRIPE)], agg_lo_out.at[pl.ds(row0, _STRIPE)])
        pltpu.sync_copy(densh.at[pl.ds(row0, _STRIPE)], den_out.at[pl.ds(row0, _STRIPE)])

    @pl.when(cid == 1)
    def _():
        pltpu.sync_copy(aggsh.at[pl.ds(row0, _STRIPE)], agg_hi_out.at[pl.ds(row0, _STRIPE)])


_k4_call = functools.partial(
    pl.kernel,
    out_type=[
        jax.ShapeDtypeStruct((NPAD, _HC), jnp.float32),
        jax.ShapeDtypeStruct((NPAD, _HC), jnp.float32),
        jax.ShapeDtypeStruct((NPAD, 8), jnp.float32),
    ],
    mesh=_MESH,
    scratch_types=[
        pltpu.VMEM_SHARED((NPAD, _HC), jnp.float32),
        pltpu.VMEM_SHARED((NPAD, 8), jnp.float32),
        pltpu.VMEM((_CH,), jnp.int32),
        pltpu.VMEM((_CH,), jnp.int32),
        pltpu.VMEM((_CH, _HC), jnp.float32),
        pltpu.VMEM((_CH, 32), jnp.float32),
        pltpu.VMEM((_CH, 8), jnp.float32),
        pltpu.SemaphoreType.DMA,
    ],
    compiler_params=pltpu.CompilerParams(use_tc_tiling_on_sc=False),
)(_k4_body)


def _k5_body(agglo_ref, agghi_ref, den_ref, wbd_ref, bflat_ref, out_ref):
    den = den_ref[...]
    dexp = jnp.tile(jnp.repeat(den, C // H, axis=1), (1, M))
    aggu = jnp.concatenate([agglo_ref[...], agghi_ref[...]], axis=1)
    agg = aggu / (dexp + 1e-9)
    out_ref[...] = jnp.dot(agg, wbd_ref[...], preferred_element_type=jnp.float32) + bflat_ref[...]


def kernel(node_feat, attn_weight, edge_dis, edge_index, atom_types, params):
    p = params
    src = edge_index[0].astype(jnp.int32)
    dst = edge_index[1].astype(jnp.int32)

    w1f, w1r = p['fce_w1'], p['rad_w1']
    ts = p['src_emb'] @ jnp.concatenate([w1f[32:160], w1r[32:160]], axis=1)
    td = p['dst_emb'] @ jnp.concatenate([w1f[160:288], w1r[160:288]], axis=1)
    w1a = jnp.concatenate([w1f[:32], w1r[:32]], axis=1)
    eye4 = jnp.eye(4, dtype=jnp.float32)
    w1a4 = jnp.kron(eye4, w1a)
    b1c4 = jnp.tile(jnp.concatenate([p['fce_b1'], p['rad_b1']]), 4).reshape(1, 128)
    mones = jnp.kron(jnp.eye(8, dtype=jnp.float32),
                     jnp.full((16, 16), 1.0 / 16.0, jnp.float32))
    g14 = jnp.tile(jnp.concatenate([p['fce_g1'], p['rad_g1']]), 4).reshape(1, 128)
    be14 = jnp.tile(jnp.concatenate([p['fce_be1'], p['rad_be1']]), 4).reshape(1, 128)
    g24 = jnp.tile(jnp.concatenate([p['fce_g2'], p['rad_g2']]), 4).reshape(1, 128)
    be24 = jnp.tile(jnp.concatenate([p['fce_be2'], p['rad_be2']]), 4).reshape(1, 128)
    w2bd = jnp.zeros((32, 32), jnp.float32).at[:16, :16].set(p['fce_w2']).at[16:, 16:].set(p['rad_w2'])
    w2bd4 = jnp.kron(eye4, w2bd)
    w3eb = jnp.zeros((32, 8), jnp.float32).at[:16, :].set(p['fce_w3'])
    w3eb4 = jnp.kron(eye4, w3eb)
    b3eb4 = jnp.tile(p['fce_b3'], 4).reshape(1, 32)
    w3wv = jnp.zeros((32, 32), jnp.float32).at[16:, :].set(p['rad_w3'])
    w3wv4 = jnp.kron(eye4, w3wv)
    b3wv4 = jnp.tile(p['rad_b3'], 4).reshape(1, 128)
    rrep = jnp.kron(jnp.eye(H, dtype=jnp.float32),
                    jnp.ones((1, C // H), jnp.float32))
    rrep4 = jnp.kron(eye4, rrep)
    pexp = jnp.kron(eye4, jnp.ones((1, 8), jnp.float32))
    ad = p['alpha_dot'] / np.sqrt(S)
    adm = (ad[:, None, :] * jnp.eye(H, dtype=jnp.float32)[:, :, None]
           ).transpose(0, 2, 1).reshape(H * S, H)
    adm4 = jnp.kron(eye4, adm)
    wpm = p['wproj'][_DEG_IDX]
    bpm = p['bproj'][_DEG_IDX]
    eye_m = jnp.eye(M, dtype=jnp.float32)
    wbd = (eye_m[:, None, :, None] * wpm[:, :, None, :]).reshape(M * C, M * C)
    bflat = bpm.reshape(1, M * C)

    flat = node_feat.reshape(N, M * C)
    flat_p = jnp.pad(flat, ((0, NPAD - N), (0, 0)))
    at_p = jnp.pad(atom_types.astype(jnp.int32), (0, NPAD - N)).reshape(NB_N, 1, 512)

    q, k, ps, pd = pl.pallas_call(
        _k1_body,
        grid=(NB_N,),
        in_specs=[
            pl.BlockSpec((512, M * C), lambda i: (i, 0)),
            pl.BlockSpec((1, 1, 512), lambda i: (i, 0, 0)),
            pl.BlockSpec((M * C, H * S), lambda i: (0, 0)),
            pl.BlockSpec((1, H * S), lambda i: (0, 0)),
            pl.BlockSpec((M * C, H * S), lambda i: (0, 0)),
            pl.BlockSpec((1, H * S), lambda i: (0, 0)),
            pl.BlockSpec((256, 32), lambda i: (0, 0)),
            pl.BlockSpec((256, 32), lambda i: (0, 0)),
        ],
        out_specs=[
            pl.BlockSpec((512, H * S), lambda i: (i, 0)),
            pl.BlockSpec((512, H * S), lambda i: (i, 0)),
            pl.BlockSpec((512, 32), lambda i: (i, 0)),
            pl.BlockSpec((512, 32), lambda i: (i, 0)),
        ],
        out_shape=[
            jax.ShapeDtypeStruct((NPAD, H * S), jnp.float32),
            jax.ShapeDtypeStruct((NPAD, H * S), jnp.float32),
            jax.ShapeDtypeStruct((NPAD, 32), jnp.float32),
            jax.ShapeDtypeStruct((NPAD, 32), jnp.float32),
        ],
    )(flat_p, at_p, p['wq'], p['bq'].reshape(1, -1), p['wk'], p['bk'].reshape(1, -1), ts, td)

    pse, pde = _k2b_call(ps, pd, src, dst)
    qd, ks = _k2_call(q, k, src, dst)

    pedge4 = (p['poly'][0] + p['poly'][1] * edge_dis).reshape(E // 4, 4)
    aw4 = attn_weight.reshape(E // 4, 128)
    pse4 = pse.reshape(E // 4, 128)
    pde4 = pde.reshape(E // 4, 128)
    qd4 = qd.reshape(E // 4, 4 * H * S)
    ks4 = ks.reshape(E // 4, 4 * H * S)
    ex4, coeff4 = pl.pallas_call(
        _k3_body,
        grid=(NB_E,),
        in_specs=[
            pl.BlockSpec((EB // 4, 128), lambda i: (i, 0)),
            pl.BlockSpec((EB // 4, 128), lambda i: (i, 0)),
            pl.BlockSpec((EB // 4, 128), lambda i: (i, 0)),
            pl.BlockSpec((EB // 4, 4 * H * S), lambda i: (i, 0)),
            pl.BlockSpec((EB // 4, 4 * H * S), lambda i: (i, 0)),
            pl.BlockSpec((EB // 4, 4), lambda i: (i, 0)),
            pl.BlockSpec((128, 128), lambda i: (0, 0)),
            pl.BlockSpec((1, 128), lambda i: (0, 0)),
            pl.BlockSpec((128, 128), lambda i: (0, 0)),
            pl.BlockSpec((1, 128), lambda i: (0, 0)),
            pl.BlockSpec((1, 128), lambda i: (0, 0)),
            pl.BlockSpec((1, 128), lambda i: (0, 0)),
            pl.BlockSpec((1, 128), lambda i: (0, 0)),
            pl.BlockSpec((128, 128), lambda i: (0, 0)),
            pl.BlockSpec((128, 32), lambda i: (0, 0)),
            pl.BlockSpec((1, 32), lambda i: (0, 0)),
            pl.BlockSpec((128, 128), lambda i: (0, 0)),
            pl.BlockSpec((1, 128), lambda i: (0, 0)),
            pl.BlockSpec((4 * H * S, 32), lambda i: (0, 0)),
            pl.BlockSpec((4, 32), lambda i: (0, 0)),
            pl.BlockSpec((32, 128), lambda i: (0, 0)),
        ],
        out_specs=[
            pl.BlockSpec((EB // 4, 32), lambda i: (i, 0)),
            pl.BlockSpec((EB // 4, 128), lambda i: (i, 0)),
        ],
        out_shape=[
            jax.ShapeDtypeStruct((E // 4, 32), jnp.float32),
            jax.ShapeDtypeStruct((E // 4, 128), jnp.float32),
        ],
    )(aw4, pse4, pde4, qd4, ks4, pedge4, w1a4, b1c4, mones, g14, be14, g24, be24,
      w2bd4, w3eb4, b3eb4, w3wv4, b3wv4, adm4, pexp, rrep4)
    ex = ex4.reshape(E, 8)
    coeff = coeff4.reshape(E, 32)

    nf_lo = flat[:, :_HC]
    nf_hi = flat[:, _HC:]
    zagg = jnp.zeros((_STRIPE, _HC), jnp.float32)
    zden = jnp.zeros((_STRIPE, 8), jnp.float32)
    agg_lo, agg_hi, den = _k4_call(nf_lo, nf_hi, coeff, ex, src, dst, zagg, zden)

    out = pl.pallas_call(
        _k5_body,
        grid=(NB_N,),
        in_specs=[
            pl.BlockSpec((512, _HC), lambda i: (i, 0)),
            pl.BlockSpec((512, _HC), lambda i: (i, 0)),
            pl.BlockSpec((512, 8), lambda i: (i, 0)),
            pl.BlockSpec((M * C, M * C), lambda i: (0, 0)),
            pl.BlockSpec((1, M * C), lambda i: (0, 0)),
        ],
        out_specs=pl.BlockSpec((512, M * C), lambda i: (i, 0)),
        out_shape=jax.ShapeDtypeStruct((NPAD, M * C), jnp.float32),
    )(agg_lo, agg_hi, den, wbd, bflat)

    return out[:N].reshape(N, M, C)


# speedup vs baseline: 1.2802x; 1.2433x over previous
import functools
import jax
import jax.numpy as jnp
import numpy as np
from jax import lax
from jax.experimental import pallas as pl
from jax.experimental.pallas import tpu as pltpu
from jax.experimental.pallas import tpu_sc as plsc

N = 10000
E = 160000
C = 32
H = 8
S = 32
M = 9
NPAD = 10240
NB_N = NPAD // 512
EB = 1280
NB_E = E // EB
_DEG_IDX = np.array([0, 1, 1, 1, 2, 2, 2, 2, 2])

_NC = 2
_NS = 16
_CH = 128
_NCHUNK = E // _CH
_HC = M * C // 2
_STRIPE = NPAD // _NS

_MESH = plsc.VectorSubcoreMesh(core_axis_name="c", subcore_axis_name="s")


def _ln(x, g, b):
    mu = x.mean(-1, keepdims=True)
    var = ((x - mu) ** 2).mean(-1, keepdims=True)
    return (x - mu) / jnp.sqrt(var + 1e-5) * g + b


def _k1_body(flat_ref, at_ref, wq_ref, bq_ref, wk_ref, bk_ref, ts_ref, td_ref,
             q_ref, k_ref, ps_ref, pd_ref):
    flat = flat_ref[...]

    def pack(x):
        xb = x.astype(jnp.bfloat16)
        lo = jax.lax.bitcast_convert_type(xb[:, :128], jnp.uint16).astype(jnp.uint32)
        hi = jax.lax.bitcast_convert_type(xb[:, 128:], jnp.uint16).astype(jnp.uint32)
        return jax.lax.bitcast_convert_type((hi << 16) | lo, jnp.int32)

    q_ref[...] = pack(jnp.dot(flat, wq_ref[...], preferred_element_type=jnp.float32) + bq_ref[...])
    k_ref[...] = pack(jnp.dot(flat, wk_ref[...], preferred_element_type=jnp.float32) + bk_ref[...])
    at = at_ref[0]
    oh = (at.reshape(512, 1) == jax.lax.broadcasted_iota(jnp.int32, (512, 256), 1)).astype(jnp.float32)
    ps_ref[...] = jnp.dot(oh, ts_ref[...], preferred_element_type=jnp.float32)
    pd_ref[...] = jnp.dot(oh, td_ref[...], preferred_element_type=jnp.float32)


def _k2_body(q_hbm, k_hbm, src_hbm, dst_hbm,
             qd_out, ks_out,
             idxs, idxd, qrows, krows, sem):
    cid = lax.axis_index("c")
    sid = lax.axis_index("s")
    w = sid * _NC + cid
    nw = _NC * _NS
    niter = (_NCHUNK - 1 - w) // nw + 1

    def body(i, carry):
        base = (w + i * nw) * _CH
        pltpu.sync_copy(src_hbm.at[pl.ds(base, _CH)], idxs)
        pltpu.sync_copy(dst_hbm.at[pl.ds(base, _CH)], idxd)
        cp1 = pltpu.async_copy(q_hbm.at[idxd], qrows, sem)
        cp2 = pltpu.async_copy(k_hbm.at[idxs], krows, sem)
        cp1.wait()
        cp2.wait()
        pltpu.sync_copy(qrows, qd_out.at[pl.ds(base, _CH)])
        pltpu.sync_copy(krows, ks_out.at[pl.ds(base, _CH)])
        return carry

    lax.fori_loop(0, niter, body, 0)


_k2_call = functools.partial(
    pl.kernel,
    out_type=[
        jax.ShapeDtypeStruct((E, 128), jnp.int32),
        jax.ShapeDtypeStruct((E, 128), jnp.int32),
    ],
    mesh=_MESH,
    scratch_types=[
        pltpu.VMEM((_CH,), jnp.int32),
        pltpu.VMEM((_CH,), jnp.int32),
        pltpu.VMEM((_CH, 128), jnp.int32),
        pltpu.VMEM((_CH, 128), jnp.int32),
        pltpu.SemaphoreType.DMA,
    ],
)(_k2_body)


def _k2b_body(ps_hbm, pd_hbm, src_hbm, dst_hbm,
              pse_out, pde_out,
              idxs, idxd, psrows, pdrows, sem):
    cid = lax.axis_index("c")
    sid = lax.axis_index("s")
    w = sid * _NC + cid
    nw = _NC * _NS
    niter = (_NCHUNK - 1 - w) // nw + 1

    def body(i, carry):
        base = (w + i * nw) * _CH
        pltpu.sync_copy(src_hbm.at[pl.ds(base, _CH)], idxs)
        pltpu.sync_copy(dst_hbm.at[pl.ds(base, _CH)], idxd)
        cp1 = pltpu.async_copy(ps_hbm.at[idxs], psrows, sem)
        cp2 = pltpu.async_copy(pd_hbm.at[idxd], pdrows, sem)
        cp1.wait()
        cp2.wait()
        pltpu.sync_copy(psrows, pse_out.at[pl.ds(base, _CH)])
        pltpu.sync_copy(pdrows, pde_out.at[pl.ds(base, _CH)])
        return carry

    lax.fori_loop(0, niter, body, 0)


_k2b_call = functools.partial(
    pl.kernel,
    out_type=[
        jax.ShapeDtypeStruct((E, 32), jnp.float32),
        jax.ShapeDtypeStruct((E, 32), jnp.float32),
    ],
    mesh=_MESH,
    scratch_types=[
        pltpu.VMEM((_CH,), jnp.int32),
        pltpu.VMEM((_CH,), jnp.int32),
        pltpu.VMEM((_CH, 32), jnp.float32),
        pltpu.VMEM((_CH, 32), jnp.float32),
        pltpu.SemaphoreType.DMA,
    ],
    compiler_params=pltpu.CompilerParams(use_tc_tiling_on_sc=False),
)(_k2b_body)


def _k3_body(aw4_ref, pse4_ref, pde4_ref, qd4_ref, ks4_ref, pedge4_ref,
             w1a4_ref, b1c4_ref, mones_ref, g14_ref, be14_ref, g24_ref, be24_ref,
             w2bd4_ref, w3eb4_ref, b3eb4_ref, w3wv4_ref, b3wv4_ref,
             admlo4_ref, admhi4_ref, pexp_ref, rrep4_ref,
             ex_ref, coeff_ref):
    mones = mones_ref[...]

    def ln4(h, g, b):
        mu = jnp.dot(h, mones, preferred_element_type=jnp.float32)
        d = h - mu
        var = jnp.dot(d * d, mones, preferred_element_type=jnp.float32)
        return d * jax.lax.rsqrt(var + 1e-5) * g + b

    h1 = jnp.dot(aw4_ref[...], w1a4_ref[...], preferred_element_type=jnp.float32) \
        + pse4_ref[...] + pde4_ref[...] + b1c4_ref[...]
    s1 = jax.nn.silu(ln4(h1, g14_ref[...], be14_ref[...]))
    h2 = jnp.dot(s1, w2bd4_ref[...], preferred_element_type=jnp.float32)
    s2 = jax.nn.silu(ln4(h2, g24_ref[...], be24_ref[...]))
    eb4 = jnp.dot(s2, w3eb4_ref[...], preferred_element_type=jnp.float32) + b3eb4_ref[...]
    wv4 = jnp.dot(s2, w3wv4_ref[...], preferred_element_type=jnp.float32) + b3wv4_ref[...]
    qdw = jax.lax.bitcast_convert_type(qd4_ref[...], jnp.uint32)
    ksw = jax.lax.bitcast_convert_type(ks4_ref[...], jnp.uint32)

    def unlo(w):
        return jax.lax.bitcast_convert_type(w << 16, jnp.float32)

    def unhi(w):
        return jax.lax.bitcast_convert_type(w & jnp.uint32(0xFFFF0000), jnp.float32)

    def sleaky(qk):
        return 0.6 * qk + 0.4 * qk * (2.0 * jax.nn.sigmoid(qk) - 1.0)

    pre_lo = sleaky(unlo(qdw) * unlo(ksw))
    pre_hi = sleaky(unhi(qdw) * unhi(ksw))
    alpha4 = jnp.dot(pre_lo, admlo4_ref[...], preferred_element_type=jnp.float32) \
        + jnp.dot(pre_hi, admhi4_ref[...], preferred_element_type=jnp.float32) \
        + eb4 + jnp.dot(pedge4_ref[...], pexp_ref[...], preferred_element_type=jnp.float32)
    ex4 = jnp.exp(alpha4)
    ex_ref[...] = ex4
    coeff_ref[...] = wv4 * jnp.dot(ex4, rrep4_ref[...], preferred_element_type=jnp.float32)


def _k4_body(nf_lo, nf_hi, coeff_hbm, ex_hbm, src_hbm, dst_hbm, zagg_hbm, zden_hbm,
             agg_lo_out, agg_hi_out, den_out,
             aggsh, densh,
             idxs0, idxd0, nfrows0, coeffv0,
             exv, gsem0):
    cid = lax.axis_index("c")
    sid = lax.axis_index("s")
    pltpu.sync_copy(zagg_hbm, aggsh.at[pl.ds(sid * _STRIPE, _STRIPE)])

    @pl.when(cid == 0)
    def _():
        pltpu.sync_copy(zden_hbm, densh.at[pl.ds(sid * _STRIPE, _STRIPE)])

    plsc.subcore_barrier()

    def prefetch(c, idxs, idxd, nfrows, coeffv, gsem):
        base = c * _CH
        pltpu.sync_copy(src_hbm.at[pl.ds(base, _CH)], idxs)
        pltpu.sync_copy(dst_hbm.at[pl.ds(base, _CH)], idxd)

        @pl.when(cid == 0)
        def _():
            pltpu.async_copy(nf_lo.at[idxs], nfrows, gsem)

        @pl.when(cid == 1)
        def _():
            pltpu.async_copy(nf_hi.at[idxs], nfrows, gsem)

        pltpu.sync_copy(coeff_hbm.at[pl.ds(base, _CH)], coeffv)

    def process(c, idxs, idxd, nfrows, coeffv, gsem):
        base = c * _CH

        @pl.when(cid == 0)
        def _():
            pltpu.make_async_copy(nf_lo.at[idxs], nfrows, gsem).wait()

        @pl.when(cid == 1)
        def _():
            pltpu.make_async_copy(nf_hi.at[idxs], nfrows, gsem).wait()

        @plsc.parallel_loop(0, _CH, step=1, unroll=4)
        def _(e):
            ca = coeffv[e, pl.ds(0, 16)]
            cb = coeffv[e, pl.ds(16, 16)]
            ma = jnp.where(cid == 0, ca, cb)
            mb = jnp.where(cid == 0, cb, ca)
            for j in range(M):
                mult = ma if j % 2 == 0 else mb
                nfrows[e, pl.ds(j * 16, 16)] = nfrows[e, pl.ds(j * 16, 16)] * mult

        pltpu.sync_copy(nfrows, aggsh.at[idxd], add=True)

        @pl.when(cid == 0)
        def _():
            pltpu.sync_copy(ex_hbm.at[pl.ds(base, _CH)], exv)
            pltpu.sync_copy(exv, densh.at[idxd], add=True)

    niter = (_NCHUNK - 1 - sid) // _NS + 1

    def body(i, carry):
        c0 = sid + i * _NS
        prefetch(c0, idxs0, idxd0, nfrows0, coeffv0, gsem0)
        process(c0, idxs0, idxd0, nfrows0, coeffv0, gsem0)
        return carry

    lax.fori_loop(0, niter, body, 0)
    plsc.subcore_barrier()
    row0 = sid * _STRIPE

    @pl.when(cid == 0)
    def _():
        pltpu.sync_copy(aggsh.at[pl.ds(row0, _STRIPE)], agg_lo_out.at[pl.ds(row0, _STRIPE)])
        pltpu.sync_copy(densh.at[pl.ds(row0, _STRIPE)], den_out.at[pl.ds(row0, _STRIPE)])

    @pl.when(cid == 1)
    def _():
        pltpu.sync_copy(aggsh.at[pl.ds(row0, _STRIPE)], agg_hi_out.at[pl.ds(row0, _STRIPE)])


_k4_call = functools.partial(
    pl.kernel,
    out_type=[
        jax.ShapeDtypeStruct((NPAD, _HC), jnp.float32),
        jax.ShapeDtypeStruct((NPAD, _HC), jnp.float32),
        jax.ShapeDtypeStruct((NPAD, 8), jnp.float32),
    ],
    mesh=_MESH,
    scratch_types=[
        pltpu.VMEM_SHARED((NPAD, _HC), jnp.float32),
        pltpu.VMEM_SHARED((NPAD, 8), jnp.float32),
        pltpu.VMEM((_CH,), jnp.int32),
        pltpu.VMEM((_CH,), jnp.int32),
        pltpu.VMEM((_CH, _HC), jnp.float32),
        pltpu.VMEM((_CH, 32), jnp.float32),
        pltpu.VMEM((_CH, 8), jnp.float32),
        pltpu.SemaphoreType.DMA,
    ],
    compiler_params=pltpu.CompilerParams(use_tc_tiling_on_sc=False),
)(_k4_body)


def _k5_body(agglo_ref, agghi_ref, den_ref, wbd_ref, bflat_ref, out_ref):
    den = den_ref[...]
    dexp = jnp.tile(jnp.repeat(den, C // H, axis=1), (1, M))
    aggu = jnp.concatenate([agglo_ref[...], agghi_ref[...]], axis=1)
    agg = aggu / (dexp + 1e-9)
    out_ref[...] = jnp.dot(agg, wbd_ref[...], preferred_element_type=jnp.float32) + bflat_ref[...]


def kernel(node_feat, attn_weight, edge_dis, edge_index, atom_types, params):
    p = params
    src = edge_index[0].astype(jnp.int32)
    dst = edge_index[1].astype(jnp.int32)

    w1f, w1r = p['fce_w1'], p['rad_w1']
    ts = p['src_emb'] @ jnp.concatenate([w1f[32:160], w1r[32:160]], axis=1)
    td = p['dst_emb'] @ jnp.concatenate([w1f[160:288], w1r[160:288]], axis=1)
    w1a = jnp.concatenate([w1f[:32], w1r[:32]], axis=1)
    eye4 = jnp.eye(4, dtype=jnp.float32)
    w1a4 = jnp.kron(eye4, w1a)
    b1c4 = jnp.tile(jnp.concatenate([p['fce_b1'], p['rad_b1']]), 4).reshape(1, 128)
    mones = jnp.kron(jnp.eye(8, dtype=jnp.float32),
                     jnp.full((16, 16), 1.0 / 16.0, jnp.float32))
    g14 = jnp.tile(jnp.concatenate([p['fce_g1'], p['rad_g1']]), 4).reshape(1, 128)
    be14 = jnp.tile(jnp.concatenate([p['fce_be1'], p['rad_be1']]), 4).reshape(1, 128)
    g24 = jnp.tile(jnp.concatenate([p['fce_g2'], p['rad_g2']]), 4).reshape(1, 128)
    be24 = jnp.tile(jnp.concatenate([p['fce_be2'], p['rad_be2']]), 4).reshape(1, 128)
    w2bd = jnp.zeros((32, 32), jnp.float32).at[:16, :16].set(p['fce_w2']).at[16:, 16:].set(p['rad_w2'])
    w2bd4 = jnp.kron(eye4, w2bd)
    w3eb = jnp.zeros((32, 8), jnp.float32).at[:16, :].set(p['fce_w3'])
    w3eb4 = jnp.kron(eye4, w3eb)
    b3eb4 = jnp.tile(p['fce_b3'], 4).reshape(1, 32)
    w3wv = jnp.zeros((32, 32), jnp.float32).at[16:, :].set(p['rad_w3'])
    w3wv4 = jnp.kron(eye4, w3wv)
    b3wv4 = jnp.tile(p['rad_b3'], 4).reshape(1, 128)
    rrep = jnp.kron(jnp.eye(H, dtype=jnp.float32),
                    jnp.ones((1, C // H), jnp.float32))
    rrep4 = jnp.kron(eye4, rrep)
    pexp = jnp.kron(eye4, jnp.ones((1, 8), jnp.float32))
    ad = p['alpha_dot'] / np.sqrt(S)
    adm = (ad[:, None, :] * jnp.eye(H, dtype=jnp.float32)[:, :, None]
           ).transpose(0, 2, 1).reshape(H * S, H)
    admlo4 = jnp.kron(eye4, adm[:128])
    admhi4 = jnp.kron(eye4, adm[128:])
    wpm = p['wproj'][_DEG_IDX]
    bpm = p['bproj'][_DEG_IDX]
    eye_m = jnp.eye(M, dtype=jnp.float32)
    wbd = (eye_m[:, None, :, None] * wpm[:, :, None, :]).reshape(M * C, M * C)
    bflat = bpm.reshape(1, M * C)

    flat = node_feat.reshape(N, M * C)
    flat_p = jnp.pad(flat, ((0, NPAD - N), (0, 0)))
    at_p = jnp.pad(atom_types.astype(jnp.int32), (0, NPAD - N)).reshape(NB_N, 1, 512)

    q, k, ps, pd = pl.pallas_call(
        _k1_body,
        grid=(NB_N,),
        in_specs=[
            pl.BlockSpec((512, M * C), lambda i: (i, 0)),
            pl.BlockSpec((1, 1, 512), lambda i: (i, 0, 0)),
            pl.BlockSpec((M * C, H * S), lambda i: (0, 0)),
            pl.BlockSpec((1, H * S), lambda i: (0, 0)),
            pl.BlockSpec((M * C, H * S), lambda i: (0, 0)),
            pl.BlockSpec((1, H * S), lambda i: (0, 0)),
            pl.BlockSpec((256, 32), lambda i: (0, 0)),
            pl.BlockSpec((256, 32), lambda i: (0, 0)),
        ],
        out_specs=[
            pl.BlockSpec((512, 128), lambda i: (i, 0)),
            pl.BlockSpec((512, 128), lambda i: (i, 0)),
            pl.BlockSpec((512, 32), lambda i: (i, 0)),
            pl.BlockSpec((512, 32), lambda i: (i, 0)),
        ],
        out_shape=[
            jax.ShapeDtypeStruct((NPAD, 128), jnp.int32),
            jax.ShapeDtypeStruct((NPAD, 128), jnp.int32),
            jax.ShapeDtypeStruct((NPAD, 32), jnp.float32),
            jax.ShapeDtypeStruct((NPAD, 32), jnp.float32),
        ],
    )(flat_p, at_p, p['wq'], p['bq'].reshape(1, -1), p['wk'], p['bk'].reshape(1, -1), ts, td)

    pse, pde = _k2b_call(ps, pd, src, dst)
    qd, ks = _k2_call(q, k, src, dst)

    pedge4 = (p['poly'][0] + p['poly'][1] * edge_dis).reshape(E // 4, 4)
    aw4 = attn_weight.reshape(E // 4, 128)
    pse4 = pse.reshape(E // 4, 128)
    pde4 = pde.reshape(E // 4, 128)
    qd4 = qd.reshape(E // 4, 512)
    ks4 = ks.reshape(E // 4, 512)
    ex4, coeff4 = pl.pallas_call(
        _k3_body,
        grid=(NB_E,),
        in_specs=[
            pl.BlockSpec((EB // 4, 128), lambda i: (i, 0)),
            pl.BlockSpec((EB // 4, 128), lambda i: (i, 0)),
            pl.BlockSpec((EB // 4, 128), lambda i: (i, 0)),
            pl.BlockSpec((EB // 4, 512), lambda i: (i, 0)),
            pl.BlockSpec((EB // 4, 512), lambda i: (i, 0)),
            pl.BlockSpec((EB // 4, 4), lambda i: (i, 0)),
            pl.BlockSpec((128, 128), lambda i: (0, 0)),
            pl.BlockSpec((1, 128), lambda i: (0, 0)),
            pl.BlockSpec((128, 128), lambda i: (0, 0)),
            pl.BlockSpec((1, 128), lambda i: (0, 0)),
            pl.BlockSpec((1, 128), lambda i: (0, 0)),
            pl.BlockSpec((1, 128), lambda i: (0, 0)),
            pl.BlockSpec((1, 128), lambda i: (0, 0)),
            pl.BlockSpec((128, 128), lambda i: (0, 0)),
            pl.BlockSpec((128, 32), lambda i: (0, 0)),
            pl.BlockSpec((1, 32), lambda i: (0, 0)),
            pl.BlockSpec((128, 128), lambda i: (0, 0)),
            pl.BlockSpec((1, 128), lambda i: (0, 0)),
            pl.BlockSpec((512, 32), lambda i: (0, 0)),
            pl.BlockSpec((512, 32), lambda i: (0, 0)),
            pl.BlockSpec((4, 32), lambda i: (0, 0)),
            pl.BlockSpec((32, 128), lambda i: (0, 0)),
        ],
        out_specs=[
            pl.BlockSpec((EB // 4, 32), lambda i: (i, 0)),
            pl.BlockSpec((EB // 4, 128), lambda i: (i, 0)),
        ],
        out_shape=[
            jax.ShapeDtypeStruct((E // 4, 32), jnp.float32),
            jax.ShapeDtypeStruct((E // 4, 128), jnp.float32),
        ],
    )(aw4, pse4, pde4, qd4, ks4, pedge4, w1a4, b1c4, mones, g14, be14, g24, be24,
      w2bd4, w3eb4, b3eb4, w3wv4, b3wv4, admlo4, admhi4, pexp, rrep4)
    ex = ex4.reshape(E, 8)
    coeff = coeff4.reshape(E, 32)

    nf_lo = flat[:, :_HC]
    nf_hi = flat[:, _HC:]
    zagg = jnp.zeros((_STRIPE, _HC), jnp.float32)
    zden = jnp.zeros((_STRIPE, 8), jnp.float32)
    agg_lo, agg_hi, den = _k4_call(nf_lo, nf_hi, coeff, ex, src, dst, zagg, zden)

    out = pl.pallas_call(
        _k5_body,
        grid=(NB_N,),
        in_specs=[
            pl.BlockSpec((512, _HC), lambda i: (i, 0)),
            pl.BlockSpec((512, _HC), lambda i: (i, 0)),
            pl.BlockSpec((512, 8), lambda i: (i, 0)),
            pl.BlockSpec((M * C, M * C), lambda i: (0, 0)),
            pl.BlockSpec((1, M * C), lambda i: (0, 0)),
        ],
        out_specs=pl.BlockSpec((512, M * C), lambda i: (i, 0)),
        out_shape=jax.ShapeDtypeStruct((NPAD, M * C), jnp.float32),
    )(agg_lo, agg_hi, den, wbd, bflat)

    return out[:N].reshape(N, M, C)
